# Initial kernel scaffold; baseline (speedup 1.0000x reference)
import jax
import jax.numpy as jnp
from jax.experimental import pallas as pl


def kernel(x, w_router, w1, w2):
    raise NotImplementedError("write your pallas kernel here")


# speedup vs baseline: 3.5037x; 3.5037x over previous
import functools

import jax
import jax.numpy as jnp
from jax.experimental import pallas as pl
from jax.experimental.pallas import tpu as pltpu

HIDDEN = 768
FFN = 3072
NEXP = 64
TOKENS = 2048
SLOTS = 2 * TOKENS

RT = 256
TM = 128
BF = 512
NF = FFN // BF
RPAD = 4736


def _router_body(x_ref, wr_ref, oh1_ref, oh2_ref, roh1_ref, roh2_ref,
                 wts_ref, counts_ref):
    i = pl.program_id(0)

    @pl.when(i == 0)
    def _():
        counts_ref[...] = jnp.zeros_like(counts_ref)

    carry = counts_ref[...]
    logits = jax.lax.dot_general(
        x_ref[...], wr_ref[...], (((1,), (1,)), ((), ())),
        preferred_element_type=jnp.float32)

    lane = jax.lax.broadcasted_iota(jnp.int32, (RT, NEXP), 1)
    m1 = jnp.max(logits, axis=1, keepdims=True)
    i1 = jnp.min(jnp.where(logits == m1, lane, NEXP), axis=1, keepdims=True)
    oh1 = (lane == i1).astype(jnp.float32)
    masked = jnp.where(lane == i1, -jnp.inf, logits)
    m2 = jnp.max(masked, axis=1, keepdims=True)
    i2 = jnp.min(jnp.where(masked == m2, lane, NEXP), axis=1, keepdims=True)
    oh2 = (lane == i2).astype(jnp.float32)

    w1c = 1.0 / (1.0 + jnp.exp(m2 - m1))

    m = oh1 + oh2
    row = jax.lax.broadcasted_iota(jnp.int32, (RT, RT), 0)
    col = jax.lax.broadcasted_iota(jnp.int32, (RT, RT), 1)
    tri = (col < row).astype(jnp.float32)
    cex = jnp.dot(tri, m, preferred_element_type=jnp.float32) + carry

    oh1_ref[...] = oh1
    oh2_ref[...] = oh2
    roh1_ref[...] = cex * oh1
    roh2_ref[...] = cex * oh2
    wts_ref[...] = jnp.concatenate([w1c, 1.0 - w1c], axis=1)
    counts_ref[...] = carry + jnp.sum(m, axis=0, keepdims=True)


def _router(x2d, w_router):
    nrt = TOKENS // RT
    return pl.pallas_call(
        _router_body,
        grid=(nrt,),
        in_specs=[
            pl.BlockSpec((RT, HIDDEN), lambda i: (i, 0)),
            pl.BlockSpec((NEXP, HIDDEN), lambda i: (0, 0)),
        ],
        out_specs=[
            pl.BlockSpec((RT, NEXP), lambda i: (i, 0)),
            pl.BlockSpec((RT, NEXP), lambda i: (i, 0)),
            pl.BlockSpec((RT, NEXP), lambda i: (i, 0)),
            pl.BlockSpec((RT, NEXP), lambda i: (i, 0)),
            pl.BlockSpec((RT, 2), lambda i: (i, 0)),
            pl.BlockSpec((1, NEXP), lambda i: (0, 0)),
        ],
        out_shape=[
            jax.ShapeDtypeStruct((TOKENS, NEXP), jnp.float32),
            jax.ShapeDtypeStruct((TOKENS, NEXP), jnp.float32),
            jax.ShapeDtypeStruct((TOKENS, NEXP), jnp.float32),
            jax.ShapeDtypeStruct((TOKENS, NEXP), jnp.float32),
            jax.ShapeDtypeStruct((TOKENS, 2), jnp.float32),
            jax.ShapeDtypeStruct((1, NEXP), jnp.float32),
        ],
        compiler_params=pltpu.CompilerParams(
            dimension_semantics=("arbitrary",)),
    )(x2d, w_router)


def _positions_body(counts_ref, oh1_ref, oh2_ref, roh1_ref, roh2_ref,
                    pos_ref, offs_ref):
    counts = counts_ref[...].astype(jnp.int32)
    ca = ((counts + 7) // 8 * 8).astype(jnp.float32)
    row = jax.lax.broadcasted_iota(jnp.int32, (NEXP, NEXP), 0)
    col = jax.lax.broadcasted_iota(jnp.int32, (NEXP, NEXP), 1)
    upper = (row < col).astype(jnp.float32)
    offs = jnp.dot(ca, upper, preferred_element_type=jnp.float32)
    total = jnp.sum(ca, axis=1, keepdims=True)

    pos1 = jnp.sum(roh1_ref[...] + oh1_ref[...] * offs, axis=1, keepdims=True)
    pos2 = jnp.sum(roh2_ref[...] + oh2_ref[...] * offs, axis=1, keepdims=True)
    pos_ref[...] = jnp.concatenate([pos1, pos2], axis=1).astype(jnp.int32)

    padded = jnp.concatenate(
        [offs, total, jnp.zeros((1, 63), jnp.float32)], axis=1)
    offs_ref[...] = padded.astype(jnp.int32)


def _positions(counts, oh1, oh2, roh1, roh2):
    return pl.pallas_call(
        _positions_body,
        out_shape=[
            jax.ShapeDtypeStruct((TOKENS, 2), jnp.int32),
            jax.ShapeDtypeStruct((1, 128), jnp.int32),
        ],
    )(counts, oh1, oh2, roh1, roh2)


def _dispatch_body(pos_ref, x_ref, xs_ref):
    xs_ref[...] = x_ref[...]


def _dispatch(posf, x2d):
    grid_spec = pltpu.PrefetchScalarGridSpec(
        num_scalar_prefetch=1,
        grid=(SLOTS,),
        in_specs=[
            pl.BlockSpec((1, 1, HIDDEN), lambda j, pos: (j // 2, 0, 0))],
        out_specs=pl.BlockSpec((1, 1, HIDDEN), lambda j, pos: (pos[j], 0, 0)),
    )
    out = pl.pallas_call(
        _dispatch_body,
        grid_spec=grid_spec,
        out_shape=jax.ShapeDtypeStruct((RPAD, 1, HIDDEN), jnp.float32),
        compiler_params=pltpu.CompilerParams(
            dimension_semantics=("arbitrary",)),
    )(posf, x2d.reshape(TOKENS, 1, HIDDEN))
    return out.reshape(RPAD, HIDDEN)


def _ffn_body(offs_ref, xs_ref, w1_ref, w2_ref, ys_ref):
    f = pl.program_id(1)
    e = pl.program_id(0)
    off = offs_ref[e]
    cnt = offs_ref[e + 1] - off
    ntiles = (cnt + TM - 1) // TM
    w1b = w1_ref[0]
    w2b = w2_ref[0]

    def tile(i, _):
        r0 = pl.multiple_of(off + i * TM, 8)
        xt = xs_ref[pl.ds(r0, TM), :]
        h = jnp.dot(xt, w1b, preferred_element_type=jnp.float32)
        h = h * 0.5 * (1.0 + jax.lax.erf(h * 0.7071067811865476))
        yp = jnp.dot(h, w2b, preferred_element_type=jnp.float32)

        @pl.when(f == 0)
        def _():
            ys_ref[pl.ds(r0, TM), :] = yp

        @pl.when(f != 0)
        def _():
            ys_ref[pl.ds(r0, TM), :] = ys_ref[pl.ds(r0, TM), :] + yp

        return 0

    jax.lax.fori_loop(0, ntiles, tile, 0)


def _ffn(offs, xs, w1, w2):
    grid_spec = pltpu.PrefetchScalarGridSpec(
        num_scalar_prefetch=1,
        grid=(NEXP, NF),
        in_specs=[
            pl.BlockSpec((RPAD, HIDDEN), lambda e, f, offs: (0, 0)),
            pl.BlockSpec((1, HIDDEN, BF), lambda e, f, offs: (e, 0, f)),
            pl.BlockSpec((1, BF, HIDDEN), lambda e, f, offs: (e, f, 0)),
        ],
        out_specs=pl.BlockSpec((RPAD, HIDDEN), lambda e, f, offs: (0, 0)),
    )
    return pl.pallas_call(
        _ffn_body,
        grid_spec=grid_spec,
        out_shape=jax.ShapeDtypeStruct((RPAD, HIDDEN), jnp.float32),
        compiler_params=pltpu.CompilerParams(
            dimension_semantics=("arbitrary", "arbitrary"),
            vmem_limit_bytes=100 * 1024 * 1024),
    )(offs, xs, w1, w2)


def _combine_body(pos_ref, wts_ref, y1_ref, y2_ref, out_ref):
    w = wts_ref[...]
    out_ref[...] = (y1_ref[...] * w[:, :, 0:1]
                    + y2_ref[...] * w[:, :, 1:2])


def _combine(posf, wts, ys):
    grid_spec = pltpu.PrefetchScalarGridSpec(
        num_scalar_prefetch=1,
        grid=(TOKENS,),
        in_specs=[
            pl.BlockSpec((1, 1, 2), lambda t, pos: (t, 0, 0)),
            pl.BlockSpec((1, 1, HIDDEN), lambda t, pos: (pos[2 * t], 0, 0)),
            pl.BlockSpec(
                (1, 1, HIDDEN), lambda t, pos: (pos[2 * t + 1], 0, 0)),
        ],
        out_specs=pl.BlockSpec((1, 1, HIDDEN), lambda t, pos: (t, 0, 0)),
    )
    ys3 = ys.reshape(RPAD, 1, HIDDEN)
    out = pl.pallas_call(
        _combine_body,
        grid_spec=grid_spec,
        out_shape=jax.ShapeDtypeStruct((TOKENS, 1, HIDDEN), jnp.float32),
        compiler_params=pltpu.CompilerParams(
            dimension_semantics=("arbitrary",)),
    )(posf, wts.reshape(TOKENS, 1, 2), ys3, ys3)
    return out.reshape(TOKENS, HIDDEN)


def kernel(x, w_router, w1, w2):
    b, t, h = x.shape
    x2d = x.reshape(t, h)
    oh1, oh2, roh1, roh2, wts, counts = _router(x2d, w_router)
    pos, offs_pad = _positions(counts, oh1, oh2, roh1, roh2)
    offs = offs_pad[0, :NEXP + 1]
    posf = pos.reshape(-1)
    xs = _dispatch(posf, x2d)
    ys = _ffn(offs, xs, w1, w2)
    out = _combine(posf, wts, ys)
    return out.reshape(b, t, h)


# speedup vs baseline: 17.1794x; 4.9032x over previous
import functools

import jax
import jax.numpy as jnp
from jax.experimental import pallas as pl
from jax.experimental.pallas import tpu as pltpu
from jax.experimental.pallas import tpu_sc as plsc

HIDDEN = 768
FFN = 3072
NEXP = 64
TOKENS = 2048
SLOTS = 2 * TOKENS

RT = 256
TM = 128
BF = 512
NF = FFN // BF
RPAD = 4736


def _router_body(x_ref, wr_ref, oh1_ref, oh2_ref, roh1_ref, roh2_ref,
                 wts_ref, counts_ref):
    i = pl.program_id(0)

    @pl.when(i == 0)
    def _():
        counts_ref[...] = jnp.zeros_like(counts_ref)

    carry = counts_ref[...]
    logits = jax.lax.dot_general(
        x_ref[...], wr_ref[...], (((1,), (1,)), ((), ())),
        preferred_element_type=jnp.float32)

    lane = jax.lax.broadcasted_iota(jnp.int32, (RT, NEXP), 1)
    m1 = jnp.max(logits, axis=1, keepdims=True)
    i1 = jnp.min(jnp.where(logits == m1, lane, NEXP), axis=1, keepdims=True)
    oh1 = (lane == i1).astype(jnp.float32)
    masked = jnp.where(lane == i1, -jnp.inf, logits)
    m2 = jnp.max(masked, axis=1, keepdims=True)
    i2 = jnp.min(jnp.where(masked == m2, lane, NEXP), axis=1, keepdims=True)
    oh2 = (lane == i2).astype(jnp.float32)

    w1c = 1.0 / (1.0 + jnp.exp(m2 - m1))

    m = oh1 + oh2
    row = jax.lax.broadcasted_iota(jnp.int32, (RT, RT), 0)
    col = jax.lax.broadcasted_iota(jnp.int32, (RT, RT), 1)
    tri = (col < row).astype(jnp.float32)
    cex = jnp.dot(tri, m, preferred_element_type=jnp.float32) + carry

    oh1_ref[...] = oh1
    oh2_ref[...] = oh2
    roh1_ref[...] = cex * oh1
    roh2_ref[...] = cex * oh2
    wts_ref[...] = jnp.concatenate([w1c, 1.0 - w1c], axis=1)
    counts_ref[...] = carry + jnp.sum(m, axis=0, keepdims=True)


def _router(x2d, w_router):
    nrt = TOKENS // RT
    return pl.pallas_call(
        _router_body,
        grid=(nrt,),
        in_specs=[
            pl.BlockSpec((RT, HIDDEN), lambda i: (i, 0)),
            pl.BlockSpec((NEXP, HIDDEN), lambda i: (0, 0)),
        ],
        out_specs=[
            pl.BlockSpec((RT, NEXP), lambda i: (i, 0)),
            pl.BlockSpec((RT, NEXP), lambda i: (i, 0)),
            pl.BlockSpec((RT, NEXP), lambda i: (i, 0)),
            pl.BlockSpec((RT, NEXP), lambda i: (i, 0)),
            pl.BlockSpec((RT, 2), lambda i: (i, 0)),
            pl.BlockSpec((1, NEXP), lambda i: (0, 0)),
        ],
        out_shape=[
            jax.ShapeDtypeStruct((TOKENS, NEXP), jnp.float32),
            jax.ShapeDtypeStruct((TOKENS, NEXP), jnp.float32),
            jax.ShapeDtypeStruct((TOKENS, NEXP), jnp.float32),
            jax.ShapeDtypeStruct((TOKENS, NEXP), jnp.float32),
            jax.ShapeDtypeStruct((TOKENS, 2), jnp.float32),
            jax.ShapeDtypeStruct((1, NEXP), jnp.float32),
        ],
        compiler_params=pltpu.CompilerParams(
            dimension_semantics=("arbitrary",)),
    )(x2d, w_router)


def _positions_body(counts_ref, oh1_ref, oh2_ref, roh1_ref, roh2_ref,
                    pos_ref, offs_ref):
    counts = counts_ref[...].astype(jnp.int32)
    ca = ((counts + 7) // 8 * 8).astype(jnp.float32)
    row = jax.lax.broadcasted_iota(jnp.int32, (NEXP, NEXP), 0)
    col = jax.lax.broadcasted_iota(jnp.int32, (NEXP, NEXP), 1)
    upper = (row < col).astype(jnp.float32)
    offs = jnp.dot(ca, upper, preferred_element_type=jnp.float32)
    total = jnp.sum(ca, axis=1, keepdims=True)

    pos1 = jnp.sum(roh1_ref[...] + oh1_ref[...] * offs, axis=1, keepdims=True)
    pos2 = jnp.sum(roh2_ref[...] + oh2_ref[...] * offs, axis=1, keepdims=True)
    pos_ref[...] = jnp.concatenate([pos1, pos2], axis=1).astype(jnp.int32)

    padded = jnp.concatenate(
        [offs, total, jnp.zeros((1, 63), jnp.float32)], axis=1)
    offs_ref[...] = padded.astype(jnp.int32)


def _positions(counts, oh1, oh2, roh1, roh2):
    return pl.pallas_call(
        _positions_body,
        out_shape=[
            jax.ShapeDtypeStruct((TOKENS, 2), jnp.int32),
            jax.ShapeDtypeStruct((1, 128), jnp.int32),
        ],
    )(counts, oh1, oh2, roh1, roh2)


NWORK = 32
TPW = TOKENS // NWORK


def _sc_dispatch_body(x_hbm, p1_hbm, p2_hbm, xs_hbm, rows_v, i1_v, i2_v, sem):
    wid = jax.lax.axis_index("s") * 2 + jax.lax.axis_index("c")
    base = wid * TPW
    pltpu.sync_copy(x_hbm.at[pl.ds(base, TPW)], rows_v)
    pltpu.sync_copy(p1_hbm.at[pl.ds(base, TPW)], i1_v)
    pltpu.sync_copy(p2_hbm.at[pl.ds(base, TPW)], i2_v)
    c1 = pltpu.make_async_copy(rows_v, xs_hbm.at[i1_v], sem)
    c2 = pltpu.make_async_copy(rows_v, xs_hbm.at[i2_v], sem)
    c1.start()
    c2.start()
    c1.wait()
    c2.wait()


def _dispatch(pos1, pos2, x2d):
    mesh = plsc.VectorSubcoreMesh(core_axis_name="c", subcore_axis_name="s")
    fn = pl.kernel(
        _sc_dispatch_body,
        out_type=jax.ShapeDtypeStruct((RPAD, HIDDEN), jnp.float32),
        mesh=mesh,
        scratch_types=[
            pltpu.VMEM((TPW, HIDDEN), jnp.float32),
            pltpu.VMEM((TPW,), jnp.int32),
            pltpu.VMEM((TPW,), jnp.int32),
            pltpu.SemaphoreType.DMA,
        ],
    )
    return fn(x2d, pos1, pos2)


def _ffn_body(offs_ref, xs_ref, w1_ref, w2_ref, ys_ref):
    f = pl.program_id(1)
    e = pl.program_id(0)
    off = offs_ref[e]
    cnt = offs_ref[e + 1] - off
    ntiles = (cnt + TM - 1) // TM
    w1b = w1_ref[0]
    w2b = w2_ref[0]

    def tile(i, _):
        r0 = pl.multiple_of(off + i * TM, 8)
        xt = xs_ref[pl.ds(r0, TM), :]
        h = jnp.dot(xt, w1b, preferred_element_type=jnp.float32)
        h = h * 0.5 * (1.0 + jax.lax.erf(h * 0.7071067811865476))
        yp = jnp.dot(h, w2b, preferred_element_type=jnp.float32)

        @pl.when(f == 0)
        def _():
            ys_ref[pl.ds(r0, TM), :] = yp

        @pl.when(f != 0)
        def _():
            ys_ref[pl.ds(r0, TM), :] = ys_ref[pl.ds(r0, TM), :] + yp

        return 0

    jax.lax.fori_loop(0, ntiles, tile, 0)


def _ffn(offs, xs, w1, w2):
    grid_spec = pltpu.PrefetchScalarGridSpec(
        num_scalar_prefetch=1,
        grid=(NEXP, NF),
        in_specs=[
            pl.BlockSpec((RPAD, HIDDEN), lambda e, f, offs: (0, 0)),
            pl.BlockSpec((1, HIDDEN, BF), lambda e, f, offs: (e, 0, f)),
            pl.BlockSpec((1, BF, HIDDEN), lambda e, f, offs: (e, f, 0)),
        ],
        out_specs=pl.BlockSpec((RPAD, HIDDEN), lambda e, f, offs: (0, 0)),
    )
    return pl.pallas_call(
        _ffn_body,
        grid_spec=grid_spec,
        out_shape=jax.ShapeDtypeStruct((RPAD, HIDDEN), jnp.float32),
        compiler_params=pltpu.CompilerParams(
            dimension_semantics=("arbitrary", "arbitrary"),
            vmem_limit_bytes=100 * 1024 * 1024),
    )(offs, xs, w1, w2)


def _sc_combine_body(ys_hbm, p1_hbm, p2_hbm, w1_hbm, w2_hbm, out_hbm,
                     r1_v, r2_v, i1_v, i2_v, w1_v, w2_v, sem):
    wid = jax.lax.axis_index("s") * 2 + jax.lax.axis_index("c")
    base = wid * TPW
    pltpu.sync_copy(p1_hbm.at[pl.ds(base, TPW)], i1_v)
    pltpu.sync_copy(p2_hbm.at[pl.ds(base, TPW)], i2_v)
    pltpu.sync_copy(w1_hbm.at[pl.ds(base, TPW)], w1_v)
    pltpu.sync_copy(w2_hbm.at[pl.ds(base, TPW)], w2_v)
    g1 = pltpu.make_async_copy(ys_hbm.at[i1_v], r1_v, sem)
    g2 = pltpu.make_async_copy(ys_hbm.at[i2_v], r2_v, sem)
    g1.start()
    g2.start()
    g1.wait()
    g2.wait()

    @pl.loop(0, TPW)
    def _tok(i):
        w1v = w1_v[i, :]
        w2v = w2_v[i, :]
        for c in range(HIDDEN // 16):
            sl = pl.ds(c * 16, 16)
            r1_v[i, sl] = r1_v[i, sl] * w1v + r2_v[i, sl] * w2v

    pltpu.sync_copy(r1_v, out_hbm.at[pl.ds(base, TPW)])


def _combine(pos1, pos2, wts, ys):
    mesh = plsc.VectorSubcoreMesh(core_axis_name="c", subcore_axis_name="s")
    fn = pl.kernel(
        _sc_combine_body,
        out_type=jax.ShapeDtypeStruct((TOKENS, HIDDEN), jnp.float32),
        mesh=mesh,
        scratch_types=[
            pltpu.VMEM((TPW, HIDDEN), jnp.float32),
            pltpu.VMEM((TPW, HIDDEN), jnp.float32),
            pltpu.VMEM((TPW,), jnp.int32),
            pltpu.VMEM((TPW,), jnp.int32),
            pltpu.VMEM((TPW, 16), jnp.float32),
            pltpu.VMEM((TPW, 16), jnp.float32),
            pltpu.SemaphoreType.DMA,
        ],
    )
    w1bc = jnp.broadcast_to(wts[:, 0:1], (TOKENS, 16))
    w2bc = jnp.broadcast_to(wts[:, 1:2], (TOKENS, 16))
    return fn(ys, pos1, pos2, w1bc, w2bc)


def kernel(x, w_router, w1, w2):
    b, t, h = x.shape
    x2d = x.reshape(t, h)
    oh1, oh2, roh1, roh2, wts, counts = _router(x2d, w_router)
    pos, offs_pad = _positions(counts, oh1, oh2, roh1, roh2)
    offs = offs_pad[0, :NEXP + 1]
    pos1 = pos[:, 0]
    pos2 = pos[:, 1]
    xs = _dispatch(pos1, pos2, x2d)
    ys = _ffn(offs, xs, w1, w2)
    out = _combine(pos1, pos2, wts, ys)
    return out.reshape(b, t, h)


# speedup vs baseline: 21.9362x; 1.2769x over previous
import functools

import jax
import jax.numpy as jnp
from jax.experimental import pallas as pl
from jax.experimental.pallas import tpu as pltpu
from jax.experimental.pallas import tpu_sc as plsc

HIDDEN = 768
FFN = 3072
NEXP = 64
TOKENS = 2048
SLOTS = 2 * TOKENS

RT = 256
TM = 128
BF = 1024
NF = FFN // BF
RPAD = 4736


def _router_body(x_ref, wr_ref, oh1_ref, oh2_ref, roh1_ref, roh2_ref,
                 wts_ref, counts_ref):
    i = pl.program_id(0)

    @pl.when(i == 0)
    def _():
        counts_ref[...] = jnp.zeros_like(counts_ref)

    carry = counts_ref[...]
    logits = jax.lax.dot_general(
        x_ref[...], wr_ref[...], (((1,), (1,)), ((), ())),
        preferred_element_type=jnp.float32)

    lane = jax.lax.broadcasted_iota(jnp.int32, (RT, NEXP), 1)
    m1 = jnp.max(logits, axis=1, keepdims=True)
    i1 = jnp.min(jnp.where(logits == m1, lane, NEXP), axis=1, keepdims=True)
    oh1 = (lane == i1).astype(jnp.float32)
    masked = jnp.where(lane == i1, -jnp.inf, logits)
    m2 = jnp.max(masked, axis=1, keepdims=True)
    i2 = jnp.min(jnp.where(masked == m2, lane, NEXP), axis=1, keepdims=True)
    oh2 = (lane == i2).astype(jnp.float32)

    w1c = 1.0 / (1.0 + jnp.exp(m2 - m1))

    m = oh1 + oh2
    row = jax.lax.broadcasted_iota(jnp.int32, (RT, RT), 0)
    col = jax.lax.broadcasted_iota(jnp.int32, (RT, RT), 1)
    tri = (col < row).astype(jnp.float32)
    cex = jnp.dot(tri, m, preferred_element_type=jnp.float32) + carry

    oh1_ref[...] = oh1
    oh2_ref[...] = oh2
    roh1_ref[...] = cex * oh1
    roh2_ref[...] = cex * oh2
    wts_ref[...] = jnp.concatenate([w1c, 1.0 - w1c], axis=1)
    counts_ref[...] = carry + jnp.sum(m, axis=0, keepdims=True)


def _router(x2d, w_router):
    nrt = TOKENS // RT
    return pl.pallas_call(
        _router_body,
        grid=(nrt,),
        in_specs=[
            pl.BlockSpec((RT, HIDDEN), lambda i: (i, 0)),
            pl.BlockSpec((NEXP, HIDDEN), lambda i: (0, 0)),
        ],
        out_specs=[
            pl.BlockSpec((RT, NEXP), lambda i: (i, 0)),
            pl.BlockSpec((RT, NEXP), lambda i: (i, 0)),
            pl.BlockSpec((RT, NEXP), lambda i: (i, 0)),
            pl.BlockSpec((RT, NEXP), lambda i: (i, 0)),
            pl.BlockSpec((RT, 2), lambda i: (i, 0)),
            pl.BlockSpec((1, NEXP), lambda i: (0, 0)),
        ],
        out_shape=[
            jax.ShapeDtypeStruct((TOKENS, NEXP), jnp.float32),
            jax.ShapeDtypeStruct((TOKENS, NEXP), jnp.float32),
            jax.ShapeDtypeStruct((TOKENS, NEXP), jnp.float32),
            jax.ShapeDtypeStruct((TOKENS, NEXP), jnp.float32),
            jax.ShapeDtypeStruct((TOKENS, 2), jnp.float32),
            jax.ShapeDtypeStruct((1, NEXP), jnp.float32),
        ],
        compiler_params=pltpu.CompilerParams(
            dimension_semantics=("arbitrary",)),
    )(x2d, w_router)


def _positions_body(counts_ref, oh1_ref, oh2_ref, roh1_ref, roh2_ref,
                    pos_ref, offs_ref):
    counts = counts_ref[...].astype(jnp.int32)
    ca = ((counts + 7) // 8 * 8).astype(jnp.float32)
    row = jax.lax.broadcasted_iota(jnp.int32, (NEXP, NEXP), 0)
    col = jax.lax.broadcasted_iota(jnp.int32, (NEXP, NEXP), 1)
    upper = (row < col).astype(jnp.float32)
    offs = jnp.dot(ca, upper, preferred_element_type=jnp.float32)
    total = jnp.sum(ca, axis=1, keepdims=True)

    pos1 = jnp.sum(roh1_ref[...] + oh1_ref[...] * offs, axis=1, keepdims=True)
    pos2 = jnp.sum(roh2_ref[...] + oh2_ref[...] * offs, axis=1, keepdims=True)
    pos_ref[...] = jnp.concatenate([pos1, pos2], axis=1).astype(jnp.int32)

    padded = jnp.concatenate(
        [offs, total, jnp.zeros((1, 63), jnp.float32)], axis=1)
    offs_ref[...] = padded.astype(jnp.int32)


def _positions(counts, oh1, oh2, roh1, roh2):
    return pl.pallas_call(
        _positions_body,
        out_shape=[
            jax.ShapeDtypeStruct((TOKENS, 2), jnp.int32),
            jax.ShapeDtypeStruct((1, 128), jnp.int32),
        ],
    )(counts, oh1, oh2, roh1, roh2)


NWORK = 32
TPW = TOKENS // NWORK


def _sc_dispatch_body(x_hbm, p1_hbm, p2_hbm, xs_hbm, rows_v, i1_v, i2_v, sem):
    wid = jax.lax.axis_index("s") * 2 + jax.lax.axis_index("c")
    base = wid * TPW
    pltpu.sync_copy(x_hbm.at[pl.ds(base, TPW)], rows_v)
    pltpu.sync_copy(p1_hbm.at[pl.ds(base, TPW)], i1_v)
    pltpu.sync_copy(p2_hbm.at[pl.ds(base, TPW)], i2_v)
    c1 = pltpu.make_async_copy(rows_v, xs_hbm.at[i1_v], sem)
    c2 = pltpu.make_async_copy(rows_v, xs_hbm.at[i2_v], sem)
    c1.start()
    c2.start()
    c1.wait()
    c2.wait()


def _dispatch(pos1, pos2, x2d):
    mesh = plsc.VectorSubcoreMesh(core_axis_name="c", subcore_axis_name="s")
    fn = pl.kernel(
        _sc_dispatch_body,
        out_type=jax.ShapeDtypeStruct((RPAD, HIDDEN), jnp.float32),
        mesh=mesh,
        scratch_types=[
            pltpu.VMEM((TPW, HIDDEN), jnp.float32),
            pltpu.VMEM((TPW,), jnp.int32),
            pltpu.VMEM((TPW,), jnp.int32),
            pltpu.SemaphoreType.DMA,
        ],
    )
    return fn(x2d, pos1, pos2)


def _ffn_body(offs_ref, xs_ref, w1_ref, w2_ref, ys_ref):
    f = pl.program_id(1)
    e = pl.program_id(0)
    off = offs_ref[e]
    cnt = offs_ref[e + 1] - off
    ntiles = (cnt + TM - 1) // TM
    w1b = w1_ref[0]
    w2b = w2_ref[0]

    def tile(i, _):
        r0 = pl.multiple_of(off + i * TM, 8)
        xt = xs_ref[pl.ds(r0, TM), :]
        h = jnp.dot(xt, w1b, preferred_element_type=jnp.float32)
        h = h * 0.5 * (1.0 + jax.lax.erf(h * 0.7071067811865476))
        yp = jnp.dot(h, w2b, preferred_element_type=jnp.float32)

        @pl.when(f == 0)
        def _():
            ys_ref[pl.ds(r0, TM), :] = yp

        @pl.when(f != 0)
        def _():
            ys_ref[pl.ds(r0, TM), :] = ys_ref[pl.ds(r0, TM), :] + yp

        return 0

    jax.lax.fori_loop(0, ntiles, tile, 0)


def _ffn(offs, xs, w1, w2):
    grid_spec = pltpu.PrefetchScalarGridSpec(
        num_scalar_prefetch=1,
        grid=(NEXP, NF),
        in_specs=[
            pl.BlockSpec((RPAD, HIDDEN), lambda e, f, offs: (0, 0)),
            pl.BlockSpec((1, HIDDEN, BF), lambda e, f, offs: (e, 0, f)),
            pl.BlockSpec((1, BF, HIDDEN), lambda e, f, offs: (e, f, 0)),
        ],
        out_specs=pl.BlockSpec((RPAD, HIDDEN), lambda e, f, offs: (0, 0)),
    )
    return pl.pallas_call(
        _ffn_body,
        grid_spec=grid_spec,
        out_shape=jax.ShapeDtypeStruct((RPAD, HIDDEN), jnp.float32),
        compiler_params=pltpu.CompilerParams(
            dimension_semantics=("arbitrary", "arbitrary"),
            vmem_limit_bytes=100 * 1024 * 1024),
    )(offs, xs, w1, w2)


def _sc_combine_body(ys_hbm, p1_hbm, p2_hbm, w1_hbm, w2_hbm, out_hbm,
                     r1_v, r2_v, i1_v, i2_v, w1_v, w2_v, sem):
    wid = jax.lax.axis_index("s") * 2 + jax.lax.axis_index("c")
    base = wid * TPW
    pltpu.sync_copy(p1_hbm.at[pl.ds(base, TPW)], i1_v)
    pltpu.sync_copy(p2_hbm.at[pl.ds(base, TPW)], i2_v)
    pltpu.sync_copy(w1_hbm.at[pl.ds(base, TPW)], w1_v)
    pltpu.sync_copy(w2_hbm.at[pl.ds(base, TPW)], w2_v)
    g1 = pltpu.make_async_copy(ys_hbm.at[i1_v], r1_v, sem)
    g2 = pltpu.make_async_copy(ys_hbm.at[i2_v], r2_v, sem)
    g1.start()
    g2.start()
    g1.wait()
    g2.wait()

    @pl.loop(0, TPW)
    def _tok(i):
        w1v = w1_v[i, :]
        w2v = w2_v[i, :]
        for c in range(HIDDEN // 16):
            sl = pl.ds(c * 16, 16)
            r1_v[i, sl] = r1_v[i, sl] * w1v + r2_v[i, sl] * w2v

    pltpu.sync_copy(r1_v, out_hbm.at[pl.ds(base, TPW)])


def _combine(pos1, pos2, wts, ys):
    mesh = plsc.VectorSubcoreMesh(core_axis_name="c", subcore_axis_name="s")
    fn = pl.kernel(
        _sc_combine_body,
        out_type=jax.ShapeDtypeStruct((TOKENS, HIDDEN), jnp.float32),
        mesh=mesh,
        scratch_types=[
            pltpu.VMEM((TPW, HIDDEN), jnp.float32),
            pltpu.VMEM((TPW, HIDDEN), jnp.float32),
            pltpu.VMEM((TPW,), jnp.int32),
            pltpu.VMEM((TPW,), jnp.int32),
            pltpu.VMEM((TPW, 16), jnp.float32),
            pltpu.VMEM((TPW, 16), jnp.float32),
            pltpu.SemaphoreType.DMA,
        ],
    )
    w1bc = jnp.broadcast_to(wts[:, 0:1], (TOKENS, 16))
    w2bc = jnp.broadcast_to(wts[:, 1:2], (TOKENS, 16))
    return fn(ys, pos1, pos2, w1bc, w2bc)


def kernel(x, w_router, w1, w2):
    b, t, h = x.shape
    x2d = x.reshape(t, h)
    oh1, oh2, roh1, roh2, wts, counts = _router(x2d, w_router)
    pos, offs_pad = _positions(counts, oh1, oh2, roh1, roh2)
    offs = offs_pad[0, :NEXP + 1]
    pos1 = pos[:, 0]
    pos2 = pos[:, 1]
    xs = _dispatch(pos1, pos2, x2d)
    ys = _ffn(offs, xs, w1, w2)
    out = _combine(pos1, pos2, wts, ys)
    return out.reshape(b, t, h)


# speedup vs baseline: 23.9983x; 1.0940x over previous
import functools

import jax
import jax.numpy as jnp
from jax.experimental import pallas as pl
from jax.experimental.pallas import tpu as pltpu
from jax.experimental.pallas import tpu_sc as plsc

HIDDEN = 768
FFN = 3072
NEXP = 64
TOKENS = 2048
SLOTS = 2 * TOKENS

RT = 256
TM = 128
BF = 1536
NF = FFN // BF
RPAD = 4736


def _router_body(x_ref, wr_ref, oh1_ref, oh2_ref, roh1_ref, roh2_ref,
                 wts_ref, counts_ref):
    i = pl.program_id(0)

    @pl.when(i == 0)
    def _():
        counts_ref[...] = jnp.zeros_like(counts_ref)

    carry = counts_ref[...]
    logits = jax.lax.dot_general(
        x_ref[...], wr_ref[...], (((1,), (1,)), ((), ())),
        preferred_element_type=jnp.float32)

    lane = jax.lax.broadcasted_iota(jnp.int32, (RT, NEXP), 1)
    m1 = jnp.max(logits, axis=1, keepdims=True)
    i1 = jnp.min(jnp.where(logits == m1, lane, NEXP), axis=1, keepdims=True)
    oh1 = (lane == i1).astype(jnp.float32)
    masked = jnp.where(lane == i1, -jnp.inf, logits)
    m2 = jnp.max(masked, axis=1, keepdims=True)
    i2 = jnp.min(jnp.where(masked == m2, lane, NEXP), axis=1, keepdims=True)
    oh2 = (lane == i2).astype(jnp.float32)

    w1c = 1.0 / (1.0 + jnp.exp(m2 - m1))

    m = oh1 + oh2
    row = jax.lax.broadcasted_iota(jnp.int32, (RT, RT), 0)
    col = jax.lax.broadcasted_iota(jnp.int32, (RT, RT), 1)
    tri = (col < row).astype(jnp.float32)
    cex = jnp.dot(tri, m, preferred_element_type=jnp.float32) + carry

    oh1_ref[...] = oh1
    oh2_ref[...] = oh2
    roh1_ref[...] = cex * oh1
    roh2_ref[...] = cex * oh2
    wts_ref[...] = jnp.concatenate([w1c, 1.0 - w1c], axis=1)
    counts_ref[...] = carry + jnp.sum(m, axis=0, keepdims=True)


def _router(x2d, w_router):
    nrt = TOKENS // RT
    return pl.pallas_call(
        _router_body,
        grid=(nrt,),
        in_specs=[
            pl.BlockSpec((RT, HIDDEN), lambda i: (i, 0)),
            pl.BlockSpec((NEXP, HIDDEN), lambda i: (0, 0)),
        ],
        out_specs=[
            pl.BlockSpec((RT, NEXP), lambda i: (i, 0)),
            pl.BlockSpec((RT, NEXP), lambda i: (i, 0)),
            pl.BlockSpec((RT, NEXP), lambda i: (i, 0)),
            pl.BlockSpec((RT, NEXP), lambda i: (i, 0)),
            pl.BlockSpec((RT, 2), lambda i: (i, 0)),
            pl.BlockSpec((1, NEXP), lambda i: (0, 0)),
        ],
        out_shape=[
            jax.ShapeDtypeStruct((TOKENS, NEXP), jnp.float32),
            jax.ShapeDtypeStruct((TOKENS, NEXP), jnp.float32),
            jax.ShapeDtypeStruct((TOKENS, NEXP), jnp.float32),
            jax.ShapeDtypeStruct((TOKENS, NEXP), jnp.float32),
            jax.ShapeDtypeStruct((TOKENS, 2), jnp.float32),
            jax.ShapeDtypeStruct((1, NEXP), jnp.float32),
        ],
        compiler_params=pltpu.CompilerParams(
            dimension_semantics=("arbitrary",)),
    )(x2d, w_router)


def _positions_body(counts_ref, oh1_ref, oh2_ref, roh1_ref, roh2_ref,
                    pos_ref, offs_ref):
    counts = counts_ref[...].astype(jnp.int32)
    ca = ((counts + 7) // 8 * 8).astype(jnp.float32)
    row = jax.lax.broadcasted_iota(jnp.int32, (NEXP, NEXP), 0)
    col = jax.lax.broadcasted_iota(jnp.int32, (NEXP, NEXP), 1)
    upper = (row < col).astype(jnp.float32)
    offs = jnp.dot(ca, upper, preferred_element_type=jnp.float32)
    total = jnp.sum(ca, axis=1, keepdims=True)

    pos1 = jnp.sum(roh1_ref[...] + oh1_ref[...] * offs, axis=1, keepdims=True)
    pos2 = jnp.sum(roh2_ref[...] + oh2_ref[...] * offs, axis=1, keepdims=True)
    pos_ref[...] = jnp.concatenate([pos1, pos2], axis=1).astype(jnp.int32)

    padded = jnp.concatenate(
        [offs, total, jnp.zeros((1, 63), jnp.float32)], axis=1)
    offs_ref[...] = padded.astype(jnp.int32)


def _positions(counts, oh1, oh2, roh1, roh2):
    return pl.pallas_call(
        _positions_body,
        out_shape=[
            jax.ShapeDtypeStruct((TOKENS, 2), jnp.int32),
            jax.ShapeDtypeStruct((1, 128), jnp.int32),
        ],
    )(counts, oh1, oh2, roh1, roh2)


NWORK = 32
TPW = TOKENS // NWORK


def _sc_dispatch_body(x_hbm, p1_hbm, p2_hbm, xs_hbm, rows_v, i1_v, i2_v, sem):
    wid = jax.lax.axis_index("s") * 2 + jax.lax.axis_index("c")
    base = wid * TPW
    pltpu.sync_copy(x_hbm.at[pl.ds(base, TPW)], rows_v)
    pltpu.sync_copy(p1_hbm.at[pl.ds(base, TPW)], i1_v)
    pltpu.sync_copy(p2_hbm.at[pl.ds(base, TPW)], i2_v)
    c1 = pltpu.make_async_copy(rows_v, xs_hbm.at[i1_v], sem)
    c2 = pltpu.make_async_copy(rows_v, xs_hbm.at[i2_v], sem)
    c1.start()
    c2.start()
    c1.wait()
    c2.wait()


def _dispatch(pos1, pos2, x2d):
    mesh = plsc.VectorSubcoreMesh(core_axis_name="c", subcore_axis_name="s")
    fn = pl.kernel(
        _sc_dispatch_body,
        out_type=jax.ShapeDtypeStruct((RPAD, HIDDEN), jnp.float32),
        mesh=mesh,
        scratch_types=[
            pltpu.VMEM((TPW, HIDDEN), jnp.float32),
            pltpu.VMEM((TPW,), jnp.int32),
            pltpu.VMEM((TPW,), jnp.int32),
            pltpu.SemaphoreType.DMA,
        ],
    )
    return fn(x2d, pos1, pos2)


def _ffn_body(offs_ref, xs_ref, w1_ref, w2_ref, ys_ref):
    f = pl.program_id(1)
    e = pl.program_id(0)
    off = offs_ref[e]
    cnt = offs_ref[e + 1] - off
    ntiles = (cnt + TM - 1) // TM
    w1b = w1_ref[0]
    w2b = w2_ref[0]

    def tile(i, _):
        r0 = pl.multiple_of(off + i * TM, 8)
        xt = xs_ref[pl.ds(r0, TM), :]
        h = jnp.dot(xt, w1b, preferred_element_type=jnp.float32)
        h = h * 0.5 * (1.0 + jax.lax.erf(h * 0.7071067811865476))
        yp = jnp.dot(h, w2b, preferred_element_type=jnp.float32)

        @pl.when(f == 0)
        def _():
            ys_ref[pl.ds(r0, TM), :] = yp

        @pl.when(f != 0)
        def _():
            ys_ref[pl.ds(r0, TM), :] = ys_ref[pl.ds(r0, TM), :] + yp

        return 0

    jax.lax.fori_loop(0, ntiles, tile, 0)


def _ffn(offs, xs, w1, w2):
    grid_spec = pltpu.PrefetchScalarGridSpec(
        num_scalar_prefetch=1,
        grid=(NEXP, NF),
        in_specs=[
            pl.BlockSpec((RPAD, HIDDEN), lambda e, f, offs: (0, 0)),
            pl.BlockSpec((1, HIDDEN, BF), lambda e, f, offs: (e, 0, f)),
            pl.BlockSpec((1, BF, HIDDEN), lambda e, f, offs: (e, f, 0)),
        ],
        out_specs=pl.BlockSpec((RPAD, HIDDEN), lambda e, f, offs: (0, 0)),
    )
    return pl.pallas_call(
        _ffn_body,
        grid_spec=grid_spec,
        out_shape=jax.ShapeDtypeStruct((RPAD, HIDDEN), jnp.float32),
        compiler_params=pltpu.CompilerParams(
            dimension_semantics=("arbitrary", "arbitrary"),
            vmem_limit_bytes=100 * 1024 * 1024),
    )(offs, xs, w1, w2)


def _sc_combine_body(ys_hbm, p1_hbm, p2_hbm, w1_hbm, w2_hbm, out_hbm,
                     r1_v, r2_v, i1_v, i2_v, w1_v, w2_v, sem):
    wid = jax.lax.axis_index("s") * 2 + jax.lax.axis_index("c")
    base = wid * TPW
    pltpu.sync_copy(p1_hbm.at[pl.ds(base, TPW)], i1_v)
    pltpu.sync_copy(p2_hbm.at[pl.ds(base, TPW)], i2_v)
    pltpu.sync_copy(w1_hbm.at[pl.ds(base, TPW)], w1_v)
    pltpu.sync_copy(w2_hbm.at[pl.ds(base, TPW)], w2_v)
    g1 = pltpu.make_async_copy(ys_hbm.at[i1_v], r1_v, sem)
    g2 = pltpu.make_async_copy(ys_hbm.at[i2_v], r2_v, sem)
    g1.start()
    g2.start()
    g1.wait()
    g2.wait()

    @pl.loop(0, TPW)
    def _tok(i):
        w1v = w1_v[i, :]
        w2v = w2_v[i, :]
        for c in range(HIDDEN // 16):
            sl = pl.ds(c * 16, 16)
            r1_v[i, sl] = r1_v[i, sl] * w1v + r2_v[i, sl] * w2v

    pltpu.sync_copy(r1_v, out_hbm.at[pl.ds(base, TPW)])


def _combine(pos1, pos2, wts, ys):
    mesh = plsc.VectorSubcoreMesh(core_axis_name="c", subcore_axis_name="s")
    fn = pl.kernel(
        _sc_combine_body,
        out_type=jax.ShapeDtypeStruct((TOKENS, HIDDEN), jnp.float32),
        mesh=mesh,
        scratch_types=[
            pltpu.VMEM((TPW, HIDDEN), jnp.float32),
            pltpu.VMEM((TPW, HIDDEN), jnp.float32),
            pltpu.VMEM((TPW,), jnp.int32),
            pltpu.VMEM((TPW,), jnp.int32),
            pltpu.VMEM((TPW, 16), jnp.float32),
            pltpu.VMEM((TPW, 16), jnp.float32),
            pltpu.SemaphoreType.DMA,
        ],
    )
    w1bc = jnp.broadcast_to(wts[:, 0:1], (TOKENS, 16))
    w2bc = jnp.broadcast_to(wts[:, 1:2], (TOKENS, 16))
    return fn(ys, pos1, pos2, w1bc, w2bc)


def kernel(x, w_router, w1, w2):
    b, t, h = x.shape
    x2d = x.reshape(t, h)
    oh1, oh2, roh1, roh2, wts, counts = _router(x2d, w_router)
    pos, offs_pad = _positions(counts, oh1, oh2, roh1, roh2)
    offs = offs_pad[0, :NEXP + 1]
    pos1 = pos[:, 0]
    pos2 = pos[:, 1]
    xs = _dispatch(pos1, pos2, x2d)
    ys = _ffn(offs, xs, w1, w2)
    out = _combine(pos1, pos2, wts, ys)
    return out.reshape(b, t, h)
